# baseline (speedup 1.0000x reference)
import functools

import jax
import jax.numpy as jnp
from jax import lax
from jax.experimental import pallas as pl
from jax.experimental.pallas import tpu as pltpu
from jax.experimental.pallas import tpu_sc as plsc

D_MODEL = 2048
N_TOK = 8192
N_EXPERTS = 8
TOP_K = 2
RANK = 16
SCALING = 32.0 / 16.0

BN = 2048
BD = 512

_SC_INFO = plsc.get_sparse_core_info()
_NC = _SC_INFO.num_cores
_NS = _SC_INFO.num_subcores
_NW = _NC * _NS
_TOK_PER_W = N_TOK // _NW
_GROUPS = _TOK_PER_W // 16


def _logits_kernel(x_ref, wr_ref, lt_ref):
    lt_ref[...] = jax.lax.dot_general(
        wr_ref[...], x_ref[...], (((1,), (1,)), ((), ())),
        preferred_element_type=jnp.float32)


def _sc_router(lt_hbm, w_hbm, lt_v, w_v):
    wid = lax.axis_index("s") * _NC + lax.axis_index("c")
    base = wid * _TOK_PER_W
    pltpu.sync_copy(lt_hbm.at[:, pl.ds(base, _TOK_PER_W)], lt_v)
    for g in range(_GROUPS):
        sl = pl.ds(g * 16, 16)
        l = [lt_v[e, sl] for e in range(N_EXPERTS)]
        m1 = l[0]
        for e in range(1, N_EXPERTS):
            m1 = jnp.maximum(m1, l[e])
        i1 = jnp.full((16,), N_EXPERTS, jnp.int32)
        for e in range(N_EXPERTS - 1, -1, -1):
            i1 = jnp.where(l[e] == m1, jnp.full((16,), e, jnp.int32), i1)
        neg = jnp.full((16,), -jnp.inf, jnp.float32)
        l2 = [jnp.where(i1 == e, neg, l[e]) for e in range(N_EXPERTS)]
        m2 = l2[0]
        for e in range(1, N_EXPERTS):
            m2 = jnp.maximum(m2, l2[e])
        i2 = jnp.full((16,), N_EXPERTS, jnp.int32)
        for e in range(N_EXPERTS - 1, -1, -1):
            i2 = jnp.where(l2[e] == m2, jnp.full((16,), e, jnp.int32), i2)
        w1 = 1.0 / (1.0 + jnp.exp(m2 - m1))
        w2 = 1.0 - w1
        zero = jnp.zeros((16,), jnp.float32)
        for e in range(N_EXPERTS):
            w_v[e, sl] = jnp.where(i1 == e, w1,
                                   jnp.where(i2 == e, w2, zero))
    pltpu.sync_copy(w_v, w_hbm.at[:, pl.ds(base, _TOK_PER_W)])


_sc_router_call = functools.partial(
    pl.kernel,
    mesh=plsc.VectorSubcoreMesh(core_axis_name="c", subcore_axis_name="s"),
    out_type=jax.ShapeDtypeStruct((N_EXPERTS, N_TOK), jnp.float32),
    scratch_types=[
        pltpu.VMEM((N_EXPERTS, _TOK_PER_W), jnp.float32),
        pltpu.VMEM((N_EXPERTS, _TOK_PER_W), jnp.float32),
    ],
)(_sc_router)


def _fused_kernel(x_ref, wb_ref, b_ref, wt_ref, aall_ref, ball_ref,
                  out_ref, aw_ref):
    @pl.when(pl.program_id(1) == 0)
    def _bottleneck():
        ex = (jax.lax.broadcasted_iota(jnp.int32, (N_EXPERTS, N_EXPERTS * RANK), 1)
              // RANK
              == jax.lax.broadcasted_iota(
                  jnp.int32, (N_EXPERTS, N_EXPERTS * RANK), 0)
              ).astype(jnp.float32)
        wexp = jax.lax.dot_general(
            wt_ref[...], ex, (((0,), (0,)), ((), ())),
            preferred_element_type=jnp.float32)
        a = jax.lax.dot_general(
            x_ref[...].astype(jnp.bfloat16),
            aall_ref[...].astype(jnp.bfloat16),
            (((1,), (1,)), ((), ())),
            preferred_element_type=jnp.float32)
        aw_ref[...] = a * (wexp * SCALING)

    acc = jax.lax.dot_general(
        x_ref[...].astype(jnp.bfloat16),
        wb_ref[...].astype(jnp.bfloat16),
        (((1,), (1,)), ((), ())),
        preferred_element_type=jnp.float32)
    acc = acc + jnp.dot(aw_ref[...].astype(jnp.bfloat16),
                        ball_ref[...].astype(jnp.bfloat16),
                        preferred_element_type=jnp.float32)
    out_ref[...] = acc + b_ref[...]


def kernel(x, W_base, b_base, W_router, lora_A, lora_B):
    a_all = lora_A.reshape(N_EXPERTS * RANK, D_MODEL)
    b_all = lora_B.transpose(0, 2, 1).reshape(N_EXPERTS * RANK, D_MODEL)
    b2 = b_base.reshape(1, D_MODEL)

    n_tiles = N_TOK // BN
    d_tiles = D_MODEL // BD

    lt = pl.pallas_call(
        _logits_kernel,
        grid=(n_tiles,),
        in_specs=[
            pl.BlockSpec((BN, D_MODEL), lambda n: (n, 0)),
            pl.BlockSpec((N_EXPERTS, D_MODEL), lambda n: (0, 0)),
        ],
        out_specs=pl.BlockSpec((N_EXPERTS, BN), lambda n: (0, n)),
        out_shape=jax.ShapeDtypeStruct((N_EXPERTS, N_TOK), jnp.float32),
        compiler_params=pltpu.CompilerParams(
            dimension_semantics=("parallel",)),
    )(x, W_router)

    wt = _sc_router_call(lt)

    return pl.pallas_call(
        _fused_kernel,
        grid=(n_tiles, d_tiles),
        in_specs=[
            pl.BlockSpec((BN, D_MODEL), lambda n, d: (n, 0)),
            pl.BlockSpec((BD, D_MODEL), lambda n, d: (d, 0)),
            pl.BlockSpec((1, BD), lambda n, d: (0, d)),
            pl.BlockSpec((N_EXPERTS, BN), lambda n, d: (0, n)),
            pl.BlockSpec((N_EXPERTS * RANK, D_MODEL),
                         lambda n, d: (0, 0)),
            pl.BlockSpec((N_EXPERTS * RANK, BD), lambda n, d: (0, d)),
        ],
        out_specs=pl.BlockSpec((BN, BD), lambda n, d: (n, d)),
        out_shape=jax.ShapeDtypeStruct((N_TOK, D_MODEL), jnp.float32),
        scratch_shapes=[pltpu.VMEM((BN, N_EXPERTS * RANK), jnp.float32)],
        compiler_params=pltpu.CompilerParams(
            dimension_semantics=("parallel", "arbitrary")),
    )(x, W_base, b2, wt, a_all, b_all)


# speedup vs baseline: 1.0225x; 1.0225x over previous
import functools

import jax
import jax.numpy as jnp
from jax import lax
from jax.experimental import pallas as pl
from jax.experimental.pallas import tpu as pltpu
from jax.experimental.pallas import tpu_sc as plsc

D_MODEL = 2048
N_TOK = 8192
N_EXPERTS = 8
TOP_K = 2
RANK = 16
SCALING = 32.0 / 16.0

BN = 2048
BD = 512

_SC_INFO = plsc.get_sparse_core_info()
_NC = _SC_INFO.num_cores
_NS = _SC_INFO.num_subcores
_NW = _NC * _NS
_TOK_PER_W = N_TOK // _NW
_GROUPS = _TOK_PER_W // 16


def _logits_a_kernel(x_ref, wr_ref, aall_ref, lt_ref, a_ref):
    xb = x_ref[...]
    lt_ref[...] = jax.lax.dot_general(
        wr_ref[...], xb, (((1,), (1,)), ((), ())),
        preferred_element_type=jnp.float32)
    a_ref[...] = jax.lax.dot_general(
        xb, aall_ref[...], (((1,), (1,)), ((), ())),
        preferred_element_type=jnp.float32)


def _sc_router(lt_hbm, w_hbm, lt_v, w_v):
    wid = lax.axis_index("s") * _NC + lax.axis_index("c")
    base = wid * _TOK_PER_W
    pltpu.sync_copy(lt_hbm.at[:, pl.ds(base, _TOK_PER_W)], lt_v)
    for g in range(_GROUPS):
        sl = pl.ds(g * 16, 16)
        l = [lt_v[e, sl] for e in range(N_EXPERTS)]
        m1 = l[0]
        for e in range(1, N_EXPERTS):
            m1 = jnp.maximum(m1, l[e])
        i1 = jnp.full((16,), N_EXPERTS, jnp.int32)
        for e in range(N_EXPERTS - 1, -1, -1):
            i1 = jnp.where(l[e] == m1, jnp.full((16,), e, jnp.int32), i1)
        neg = jnp.full((16,), -jnp.inf, jnp.float32)
        l2 = [jnp.where(i1 == e, neg, l[e]) for e in range(N_EXPERTS)]
        m2 = l2[0]
        for e in range(1, N_EXPERTS):
            m2 = jnp.maximum(m2, l2[e])
        i2 = jnp.full((16,), N_EXPERTS, jnp.int32)
        for e in range(N_EXPERTS - 1, -1, -1):
            i2 = jnp.where(l2[e] == m2, jnp.full((16,), e, jnp.int32), i2)
        w1 = 1.0 / (1.0 + jnp.exp(m2 - m1))
        w2 = 1.0 - w1
        zero = jnp.zeros((16,), jnp.float32)
        for e in range(N_EXPERTS):
            w_v[e, sl] = jnp.where(i1 == e, w1,
                                   jnp.where(i2 == e, w2, zero))
    pltpu.sync_copy(w_v, w_hbm.at[:, pl.ds(base, _TOK_PER_W)])


_sc_router_call = functools.partial(
    pl.kernel,
    mesh=plsc.VectorSubcoreMesh(core_axis_name="c", subcore_axis_name="s"),
    out_type=jax.ShapeDtypeStruct((N_EXPERTS, N_TOK), jnp.float32),
    scratch_types=[
        pltpu.VMEM((N_EXPERTS, _TOK_PER_W), jnp.float32),
        pltpu.VMEM((N_EXPERTS, _TOK_PER_W), jnp.float32),
    ],
)(_sc_router)


def _fused_kernel(x_ref, wb_ref, b_ref, wt_ref, a_ref, ball_ref,
                  out_ref, aw_ref):
    @pl.when(pl.program_id(1) == 0)
    def _gate_bottleneck():
        ex = (jax.lax.broadcasted_iota(jnp.int32, (N_EXPERTS, N_EXPERTS * RANK), 1)
              // RANK
              == jax.lax.broadcasted_iota(
                  jnp.int32, (N_EXPERTS, N_EXPERTS * RANK), 0)
              ).astype(jnp.float32)
        wexp = jax.lax.dot_general(
            wt_ref[...], ex, (((0,), (0,)), ((), ())),
            preferred_element_type=jnp.float32)
        aw_ref[...] = a_ref[...] * (wexp * SCALING)

    acc = jax.lax.dot_general(
        x_ref[...], wb_ref[...], (((1,), (1,)), ((), ())),
        preferred_element_type=jnp.float32)
    acc = acc + jnp.dot(aw_ref[...], ball_ref[...],
                        preferred_element_type=jnp.float32)
    out_ref[...] = acc + b_ref[...]


def kernel(x, W_base, b_base, W_router, lora_A, lora_B):
    a_all = lora_A.reshape(N_EXPERTS * RANK, D_MODEL)
    b_all = lora_B.transpose(0, 2, 1).reshape(N_EXPERTS * RANK, D_MODEL)
    b2 = b_base.reshape(1, D_MODEL)

    n_tiles = N_TOK // BN
    d_tiles = D_MODEL // BD

    lt, a = pl.pallas_call(
        _logits_a_kernel,
        grid=(n_tiles,),
        in_specs=[
            pl.BlockSpec((BN, D_MODEL), lambda n: (n, 0)),
            pl.BlockSpec((N_EXPERTS, D_MODEL), lambda n: (0, 0)),
            pl.BlockSpec((N_EXPERTS * RANK, D_MODEL),
                         lambda n: (0, 0)),
        ],
        out_specs=[
            pl.BlockSpec((N_EXPERTS, BN), lambda n: (0, n)),
            pl.BlockSpec((BN, N_EXPERTS * RANK), lambda n: (n, 0)),
        ],
        out_shape=[
            jax.ShapeDtypeStruct((N_EXPERTS, N_TOK), jnp.float32),
            jax.ShapeDtypeStruct((N_TOK, N_EXPERTS * RANK), jnp.float32),
        ],
        compiler_params=pltpu.CompilerParams(
            dimension_semantics=("parallel",)),
    )(x, W_router, a_all)

    wt = _sc_router_call(lt)

    return pl.pallas_call(
        _fused_kernel,
        grid=(n_tiles, d_tiles),
        in_specs=[
            pl.BlockSpec((BN, D_MODEL), lambda n, d: (n, 0)),
            pl.BlockSpec((BD, D_MODEL), lambda n, d: (d, 0)),
            pl.BlockSpec((1, BD), lambda n, d: (0, d)),
            pl.BlockSpec((N_EXPERTS, BN), lambda n, d: (0, n)),
            pl.BlockSpec((BN, N_EXPERTS * RANK), lambda n, d: (n, 0)),
            pl.BlockSpec((N_EXPERTS * RANK, BD), lambda n, d: (0, d)),
        ],
        out_specs=pl.BlockSpec((BN, BD), lambda n, d: (n, d)),
        out_shape=jax.ShapeDtypeStruct((N_TOK, D_MODEL), jnp.float32),
        scratch_shapes=[pltpu.VMEM((BN, N_EXPERTS * RANK), jnp.float32)],
        compiler_params=pltpu.CompilerParams(
            dimension_semantics=("parallel", "arbitrary")),
    )(x, W_base, b2, wt, a, b_all)
